# baseline (speedup 1.0000x reference)
import functools

import jax
import jax.numpy as jnp
from jax import lax
from jax.experimental import pallas as pl
from jax.experimental.pallas import tpu as pltpu
from jax.experimental.pallas import tpu_sc as plsc

_BATCH = 1024
_SEQ = 200
_EMBED = 64

_NC = 2
_NS = 16
_NW = _NC * _NS

_N_ROWS = _BATCH * _SEQ
_ROWS_PER_W = _N_ROWS // _NW
_CHUNK = 128
_N_CHUNKS = _ROWS_PER_W // _CHUNK
_NBUF = 5
_N_OUTER = _N_CHUNKS // _NBUF


def _gather_kernel(table_hbm, idx_hbm, out_hbm, idx_v, rows_v, gsem, osem):
    wid = lax.axis_index("s") * _NC + lax.axis_index("c")
    base = wid * _ROWS_PER_W

    pltpu.sync_copy(idx_hbm.at[wid], idx_v)

    def body(i, carry):
        gathers = []
        for b in range(_NBUF):
            j = i * _NBUF + b
            gathers.append(
                pltpu.async_copy(table_hbm.at[idx_v.at[j]], rows_v.at[b], gsem)
            )
        for g in gathers:
            g.wait()
        outs = []
        for b in range(_NBUF):
            j = i * _NBUF + b
            row0 = base + j * _CHUNK
            outs.append(
                pltpu.async_copy(rows_v.at[b], out_hbm.at[pl.ds(row0, _CHUNK)], osem)
            )
        for o in outs:
            o.wait()
        return carry

    lax.fori_loop(0, _N_OUTER, body, 0)


@functools.partial(jax.jit, static_argnames=())
def _gather(table, idx3d):
    mesh = plsc.VectorSubcoreMesh(core_axis_name="c", subcore_axis_name="s")
    run = functools.partial(
        pl.kernel,
        mesh=mesh,
        out_type=jax.ShapeDtypeStruct((_N_ROWS, _EMBED), jnp.float32),
        scratch_types=[
            pltpu.VMEM((_N_CHUNKS, _CHUNK), jnp.int32),
            pltpu.VMEM((_NBUF, _CHUNK, _EMBED), jnp.float32),
            pltpu.SemaphoreType.DMA,
            pltpu.SemaphoreType.DMA,
        ],
        compiler_params=pltpu.CompilerParams(use_tc_tiling_on_sc=False),
    )(_gather_kernel)
    return run(table, idx3d)


def kernel(input_seq, embedding_table):
    idx3d = input_seq.astype(jnp.int32).reshape(_NW, _N_CHUNKS, _CHUNK)
    out = _gather(embedding_table, idx3d)
    return out.reshape(_BATCH, _SEQ, _EMBED)
